# baseline (speedup 1.0000x reference)
import functools

import jax
import jax.numpy as jnp
from jax.experimental import pallas as pl
from jax.experimental.pallas import tpu as pltpu

N = 50000
E = 800000
IN_DIM = 128
H = 70


def _matmul_kernel(x_ref, w_ref, b_ref, o_ref):
    o_ref[...] = (
        jnp.dot(x_ref[...], w_ref[...], preferred_element_type=jnp.float32)
        + b_ref[...]
    )


def _tc_matmul(x, w, b):
    M, K = x.shape
    F = w.shape[1]
    BM = 2000
    assert M % BM == 0
    return pl.pallas_call(
        _matmul_kernel,
        grid=(M // BM,),
        in_specs=[
            pl.BlockSpec((BM, K), lambda i: (i, 0)),
            pl.BlockSpec((K, F), lambda i: (0, 0)),
            pl.BlockSpec((F,), lambda i: (0,)),
        ],
        out_specs=pl.BlockSpec((BM, F), lambda i: (i, 0)),
        out_shape=jax.ShapeDtypeStruct((M, F), jnp.float32),
    )(x, w, b)


def _bn(x, g, b, eps=1e-5):
    m = jnp.mean(x, axis=0, keepdims=True)
    v = jnp.var(x, axis=0, keepdims=True)
    return (x - m) / jnp.sqrt(v + eps) * g + b


def _layer(h, e, src, dst, sn, se, W, b, bng_h, bnb_h, bng_e, bnb_e):
    h_in, e_in = h, e
    proj = _tc_matmul(h, jnp.concatenate([W[0], W[1], W[3], W[4]], axis=1),
                      jnp.concatenate([b[0], b[1], b[3], b[4]], axis=0))
    Ah = proj[:, 0 * H:1 * H]
    Bh = proj[:, 1 * H:2 * H]
    Dh = proj[:, 2 * H:3 * H]
    Eh = proj[:, 3 * H:4 * H]
    Ce = e @ W[2] + b[2]
    e_new = Ce + Dh[src] + Eh[dst]
    sigma = jax.nn.sigmoid(e_new)
    num = jax.ops.segment_sum(sigma * Bh[src], dst, num_segments=N)
    den = jax.ops.segment_sum(sigma, dst, num_segments=N)
    h_new = Ah + num / (den + 1e-6)
    h_new = h_new * sn
    e_new2 = e_new * se
    h_new = _bn(h_new, bng_h, bnb_h)
    e_new2 = _bn(e_new2, bng_e, bnb_e)
    return h_in + jax.nn.relu(h_new), e_in + jax.nn.relu(e_new2)


def kernel(nodes_feat, edges_feat, nodes_num_norm_sqrt, edges_num_norm_sqrt,
           edge_index, emb_h_w, emb_h_b, emb_e_w, emb_e_b,
           l1_W, l1_b, l1_bng_h, l1_bnb_h, l1_bng_e, l1_bnb_e,
           lo_W, lo_b, lo_bng_h, lo_bnb_h, lo_bng_e, lo_bnb_e):
    src = edge_index[0]
    dst = edge_index[1]
    h = _tc_matmul(nodes_feat, emb_h_w, emb_h_b)
    e = edges_feat @ emb_e_w + emb_e_b
    h, e = _layer(h, e, src, dst, nodes_num_norm_sqrt, edges_num_norm_sqrt,
                  l1_W, l1_b, l1_bng_h, l1_bnb_h, l1_bng_e, l1_bnb_e)
    h, e = _layer(h, e, src, dst, nodes_num_norm_sqrt, edges_num_norm_sqrt,
                  lo_W, lo_b, lo_bng_h, lo_bnb_h, lo_bng_e, lo_bnb_e)
    return jnp.mean(h, axis=0, keepdims=True)


# speedup vs baseline: 1.1661x; 1.1661x over previous
import dataclasses
import functools

import jax
import jax.numpy as jnp
from jax import lax
from jax.experimental import pallas as pl
from jax.experimental.pallas import tpu as pltpu
from jax.experimental.pallas import tpu_sc as plsc

N = 50000
E = 800000
IN_DIM = 128
H = 70
HP = 80
PF = 16
NPASS = HP // PF
SW = 2 * PF
CH = 128
NCHUNK = E // CH
NTILE = 32
CPT = -(-NCHUNK // NTILE)
NSUB = 16
NP = 50048
RPT = NP // NSUB
BM = 2000
BE = 2000
EPS = 1e-5

_SC_CP = pltpu.CompilerParams()
if "needs_layout_passes" in pltpu.CompilerParams.__dataclass_fields__:
    _SC_CP = dataclasses.replace(_SC_CP, needs_layout_passes=False)
if "use_tc_tiling_on_sc" in pltpu.CompilerParams.__dataclass_fields__:
    _SC_CP = dataclasses.replace(_SC_CP, use_tc_tiling_on_sc=False)


def _sc_pass_l1(edge_index, xe, se, srctab, dsttab, c1p):
    mesh = plsc.VectorSubcoreMesh(core_axis_name="c", subcore_axis_name="s")

    @functools.partial(
        pl.kernel, mesh=mesh, compiler_params=_SC_CP,
        out_type=[jax.ShapeDtypeStruct((2, NP, SW), jnp.float32),
                  jax.ShapeDtypeStruct((E, PF), jnp.float32)],
        scratch_types=[
            pltpu.VMEM((CH,), jnp.int32),
            pltpu.VMEM((CH,), jnp.int32),
            pltpu.VMEM((CH, SW), jnp.float32),
            pltpu.VMEM((CH, PF), jnp.float32),
            pltpu.VMEM((CH, SW), jnp.float32),
            pltpu.VMEM((CH, PF), jnp.float32),
            pltpu.VMEM((CH,), jnp.float32),
            pltpu.VMEM((CH,), jnp.float32),
            pltpu.VMEM((PF,), jnp.float32),
            pltpu.VMEM_SHARED((NP, SW), jnp.float32),
        ])
    def k(ei, xeh, seh, st, dt, c1h, accum_out, u_out,
          isrc, idst, rsrc, rdst, outv, uv, xes, ses, c1v, acc):
        s = lax.axis_index("s")
        c = lax.axis_index("c")
        wid = s * 2 + c
        zero16 = jnp.zeros((PF,), jnp.float32)

        @pl.loop(0, CH)
        def _(j):
            outv[j, pl.ds(0, PF)] = zero16
            outv[j, pl.ds(PF, PF)] = zero16
        r0 = s * RPT
        @pl.loop(0, RPT // CH)
        def _(q):
            pltpu.sync_copy(outv, acc.at[pl.ds(r0 + q * CH, CH)])
        @pl.when(RPT % CH != 0)
        def _():
            pltpu.sync_copy(outv.at[pl.ds(0, RPT % CH)],
                            acc.at[pl.ds(r0 + (RPT // CH) * CH, RPT % CH)])
        plsc.subcore_barrier()

        pltpu.sync_copy(c1h, c1v)
        c1 = c1v[...]

        @pl.loop(0, CPT)
        def _(kk):
            cid = wid + kk * NTILE
            @pl.when(cid < NCHUNK)
            def _():
                base = cid * CH
                pltpu.sync_copy(ei.at[0, pl.ds(base, CH)], isrc)
                pltpu.sync_copy(ei.at[1, pl.ds(base, CH)], idst)
                pltpu.sync_copy(xeh.at[pl.ds(base, CH)], xes)
                pltpu.sync_copy(seh.at[pl.ds(base, CH)], ses)
                pltpu.sync_copy(st.at[isrc], rsrc)
                pltpu.sync_copy(dt.at[idst], rdst)

                @pl.loop(0, CH)
                def _(j):
                    bh = rsrc[j, pl.ds(0, PF)]
                    dh = rsrc[j, pl.ds(PF, PF)]
                    eh = rdst[j, pl.ds(0, PF)]
                    idxj = jnp.full((PF,), j, jnp.int32)
                    xv = plsc.load_gather(xes, [idxj])
                    sv = plsc.load_gather(ses, [idxj])
                    enew = xv * c1 + dh + eh
                    sig = 1.0 / (1.0 + jnp.exp(-enew))
                    uv[j, pl.ds(0, PF)] = enew * sv
                    outv[j, pl.ds(0, PF)] = sig * bh
                    outv[j, pl.ds(PF, PF)] = sig
                pltpu.sync_copy(outv, acc.at[idst], add=True)
                pltpu.sync_copy(uv, u_out.at[pl.ds(base, CH)])

        plsc.subcore_barrier()
        @pl.loop(0, RPT // CH)
        def _(q):
            pltpu.sync_copy(acc.at[pl.ds(r0 + q * CH, CH)],
                            accum_out.at[c, pl.ds(r0 + q * CH, CH)])
        @pl.when(RPT % CH != 0)
        def _():
            pltpu.sync_copy(acc.at[pl.ds(r0 + (RPT // CH) * CH, RPT % CH)],
                            accum_out.at[c, pl.ds(r0 + (RPT // CH) * CH,
                                                  RPT % CH)])

    return k(edge_index, xe, se, srctab, dsttab, c1p)


def _sc_pass_l2(edge_index, ce, srctab, dsttab):
    mesh = plsc.VectorSubcoreMesh(core_axis_name="c", subcore_axis_name="s")

    @functools.partial(
        pl.kernel, mesh=mesh, compiler_params=_SC_CP,
        out_type=jax.ShapeDtypeStruct((2, NP, SW), jnp.float32),
        scratch_types=[
            pltpu.VMEM((CH,), jnp.int32),
            pltpu.VMEM((CH,), jnp.int32),
            pltpu.VMEM((CH, SW), jnp.float32),
            pltpu.VMEM((CH, PF), jnp.float32),
            pltpu.VMEM((CH, SW), jnp.float32),
            pltpu.VMEM((CH, PF), jnp.float32),
            pltpu.VMEM_SHARED((NP, SW), jnp.float32),
        ])
    def k(ei, ceh, st, dt, accum_out,
          isrc, idst, rsrc, rdst, outv, cev, acc):
        s = lax.axis_index("s")
        c = lax.axis_index("c")
        wid = s * 2 + c
        zero16 = jnp.zeros((PF,), jnp.float32)

        @pl.loop(0, CH)
        def _(j):
            outv[j, pl.ds(0, PF)] = zero16
            outv[j, pl.ds(PF, PF)] = zero16
        r0 = s * RPT
        @pl.loop(0, RPT // CH)
        def _(q):
            pltpu.sync_copy(outv, acc.at[pl.ds(r0 + q * CH, CH)])
        @pl.when(RPT % CH != 0)
        def _():
            pltpu.sync_copy(outv.at[pl.ds(0, RPT % CH)],
                            acc.at[pl.ds(r0 + (RPT // CH) * CH, RPT % CH)])
        plsc.subcore_barrier()

        @pl.loop(0, CPT)
        def _(kk):
            cid = wid + kk * NTILE
            @pl.when(cid < NCHUNK)
            def _():
                base = cid * CH
                pltpu.sync_copy(ei.at[0, pl.ds(base, CH)], isrc)
                pltpu.sync_copy(ei.at[1, pl.ds(base, CH)], idst)
                pltpu.sync_copy(ceh.at[pl.ds(base, CH)], cev)
                pltpu.sync_copy(st.at[isrc], rsrc)
                pltpu.sync_copy(dt.at[idst], rdst)

                @pl.loop(0, CH)
                def _(j):
                    bh = rsrc[j, pl.ds(0, PF)]
                    dh = rsrc[j, pl.ds(PF, PF)]
                    eh = rdst[j, pl.ds(0, PF)]
                    cv = cev[j, pl.ds(0, PF)]
                    enew = cv + dh + eh
                    sig = 1.0 / (1.0 + jnp.exp(-enew))
                    outv[j, pl.ds(0, PF)] = sig * bh
                    outv[j, pl.ds(PF, PF)] = sig
                pltpu.sync_copy(outv, acc.at[idst], add=True)

        plsc.subcore_barrier()
        @pl.loop(0, RPT // CH)
        def _(q):
            pltpu.sync_copy(acc.at[pl.ds(r0 + q * CH, CH)],
                            accum_out.at[c, pl.ds(r0 + q * CH, CH)])
        @pl.when(RPT % CH != 0)
        def _():
            pltpu.sync_copy(acc.at[pl.ds(r0 + (RPT // CH) * CH, RPT % CH)],
                            accum_out.at[c, pl.ds(r0 + (RPT // CH) * CH,
                                                  RPT % CH)])

    return k(edge_index, ce, srctab, dsttab)


def _tc_prep(xn, Wh, bh, Wc, bc, d1):
    def body(xn_r, Wh_r, bh_r, Wc_r, bc_r, d1_r, h0_o, Ah_o, *tab_o):
        h0 = jnp.dot(xn_r[...], Wh_r[...],
                     preferred_element_type=jnp.float32) + bh_r[...]
        proj = jnp.dot(h0, Wc_r[...],
                       preferred_element_type=jnp.float32) + bc_r[...]
        h0_o[...] = h0
        Ah_o[...] = proj[:, 0:HP]
        Bh = proj[:, HP:2 * HP]
        Dh = proj[:, 2 * HP:3 * HP]
        Eh = proj[:, 3 * HP:4 * HP]
        for p in range(NPASS):
            sl = slice(PF * p, PF * p + PF)
            tab_o[p][...] = jnp.concatenate([Bh[:, sl], Dh[:, sl]], axis=1)
            tab_o[NPASS + p][...] = Eh[:, sl] + d1_r[:, sl]

    nb = N // BM
    return pl.pallas_call(
        body,
        grid=(nb,),
        in_specs=[
            pl.BlockSpec((BM, IN_DIM), lambda i: (i, 0)),
            pl.BlockSpec((IN_DIM, HP), lambda i: (0, 0)),
            pl.BlockSpec((1, HP), lambda i: (0, 0)),
            pl.BlockSpec((HP, 4 * HP), lambda i: (0, 0)),
            pl.BlockSpec((1, 4 * HP), lambda i: (0, 0)),
            pl.BlockSpec((1, HP), lambda i: (0, 0)),
        ],
        out_specs=[pl.BlockSpec((BM, HP), lambda i: (i, 0)),
                   pl.BlockSpec((BM, HP), lambda i: (i, 0))]
        + [pl.BlockSpec((BM, SW), lambda i: (i, 0)) for _ in range(NPASS)]
        + [pl.BlockSpec((BM, PF), lambda i: (i, 0)) for _ in range(NPASS)],
        out_shape=[jax.ShapeDtypeStruct((N, HP), jnp.float32),
                   jax.ShapeDtypeStruct((N, HP), jnp.float32)]
        + [jax.ShapeDtypeStruct((N, SW), jnp.float32) for _ in range(NPASS)]
        + [jax.ShapeDtypeStruct((N, PF), jnp.float32) for _ in range(NPASS)],
    )(xn, Wh, bh, Wc, bc, d1)


def _tc_node_gate(accums, Ah, sn):
    def body(*refs):
        a5 = refs[:NPASS]
        Ah_r, sn_r, tmp_o, st_o = refs[NPASS:]
        i = pl.program_id(0)
        num = jnp.concatenate(
            [a[0, :, 0:PF] + a[1, :, 0:PF] for a in a5], axis=1)
        den = jnp.concatenate(
            [a[0, :, PF:SW] + a[1, :, PF:SW] for a in a5], axis=1)
        tmp = (Ah_r[...] + num / (den + 1e-6)) * sn_r[...]
        tmp_o[...] = tmp
        @pl.when(i == 0)
        def _():
            st_o[...] = jnp.zeros((2, HP), jnp.float32)
        st_o[...] = st_o[...] + jnp.concatenate(
            [jnp.sum(tmp, 0, keepdims=True),
             jnp.sum(tmp * tmp, 0, keepdims=True)], axis=0)

    nb = N // BM
    return pl.pallas_call(
        body,
        grid=(nb,),
        in_specs=[pl.BlockSpec((2, BM, SW), lambda i: (0, i, 0))
                  for _ in range(NPASS)]
        + [pl.BlockSpec((BM, HP), lambda i: (i, 0)),
           pl.BlockSpec((BM, 1), lambda i: (i, 0))],
        out_specs=[pl.BlockSpec((BM, HP), lambda i: (i, 0)),
                   pl.BlockSpec((2, HP), lambda i: (0, 0))],
        out_shape=[jax.ShapeDtypeStruct((N, HP), jnp.float32),
                   jax.ShapeDtypeStruct((2, HP), jnp.float32)],
    )(*accums, Ah, sn)


def _tc_edge_stats(us):
    def body(*refs):
        u5 = refs[:NPASS]
        st_o = refs[NPASS]
        i = pl.program_id(0)
        u = jnp.concatenate([r[...] for r in u5], axis=1)
        @pl.when(i == 0)
        def _():
            st_o[...] = jnp.zeros((2, HP), jnp.float32)
        st_o[...] = st_o[...] + jnp.concatenate(
            [jnp.sum(u, 0, keepdims=True),
             jnp.sum(u * u, 0, keepdims=True)], axis=0)

    nb = E // BE
    return pl.pallas_call(
        body,
        grid=(nb,),
        in_specs=[pl.BlockSpec((BE, PF), lambda i: (i, 0))
                  for _ in range(NPASS)],
        out_specs=pl.BlockSpec((2, HP), lambda i: (0, 0)),
        out_shape=jax.ShapeDtypeStruct((2, HP), jnp.float32),
    )(*us)


def _tc_node_update(tmp, h0, nstats, g, b, Wc, bc, d2):
    def body(tmp_r, h0_r, st_r, g_r, b_r, Wc_r, bc_r, d2_r,
             h1_o, Ah_o, *tab_o):
        m = st_r[0:1] * (1.0 / N)
        var = st_r[1:2] * (1.0 / N) - m * m
        inv = 1.0 / jnp.sqrt(var + EPS)
        h1 = h0_r[...] + jnp.maximum(
            (tmp_r[...] - m) * inv * g_r[...] + b_r[...], 0.0)
        proj = jnp.dot(h1, Wc_r[...],
                       preferred_element_type=jnp.float32) + bc_r[...]
        h1_o[...] = h1
        Ah_o[...] = proj[:, 0:HP]
        Bh = proj[:, HP:2 * HP]
        Dh = proj[:, 2 * HP:3 * HP]
        Eh = proj[:, 3 * HP:4 * HP]
        for p in range(NPASS):
            sl = slice(PF * p, PF * p + PF)
            tab_o[p][...] = jnp.concatenate([Bh[:, sl], Dh[:, sl]], axis=1)
            tab_o[NPASS + p][...] = Eh[:, sl] + d2_r[:, sl]

    nb = N // BM
    return pl.pallas_call(
        body,
        grid=(nb,),
        in_specs=[
            pl.BlockSpec((BM, HP), lambda i: (i, 0)),
            pl.BlockSpec((BM, HP), lambda i: (i, 0)),
            pl.BlockSpec((2, HP), lambda i: (0, 0)),
            pl.BlockSpec((1, HP), lambda i: (0, 0)),
            pl.BlockSpec((1, HP), lambda i: (0, 0)),
            pl.BlockSpec((HP, 4 * HP), lambda i: (0, 0)),
            pl.BlockSpec((1, 4 * HP), lambda i: (0, 0)),
            pl.BlockSpec((1, HP), lambda i: (0, 0)),
        ],
        out_specs=[pl.BlockSpec((BM, HP), lambda i: (i, 0)),
                   pl.BlockSpec((BM, HP), lambda i: (i, 0))]
        + [pl.BlockSpec((BM, SW), lambda i: (i, 0)) for _ in range(NPASS)]
        + [pl.BlockSpec((BM, PF), lambda i: (i, 0)) for _ in range(NPASS)],
        out_shape=[jax.ShapeDtypeStruct((N, HP), jnp.float32),
                   jax.ShapeDtypeStruct((N, HP), jnp.float32)]
        + [jax.ShapeDtypeStruct((N, SW), jnp.float32) for _ in range(NPASS)]
        + [jax.ShapeDtypeStruct((N, PF), jnp.float32) for _ in range(NPASS)],
    )(tmp, h0, nstats, g, b, Wc, bc, d2)


def _tc_edge_ce(us, xe2, estats, g, b, W22, g2):
    def body(*refs):
        u5 = refs[:NPASS]
        xe_r, st_r, g_r, b_r, W_r, g2_r = refs[NPASS:NPASS + 6]
        ce_o = refs[NPASS + 6:]
        u = jnp.concatenate([r[...] for r in u5], axis=1)
        m = st_r[0:1] * (1.0 / E)
        var = st_r[1:2] * (1.0 / E) - m * m
        inv = 1.0 / jnp.sqrt(var + EPS)
        r = jnp.maximum((u - m) * inv * g_r[...] + b_r[...], 0.0)
        ce = jnp.dot(r, W_r[...],
                     preferred_element_type=jnp.float32) + xe_r[...] * g2_r[...]
        for p in range(NPASS):
            ce_o[p][...] = ce[:, PF * p:PF * p + PF]

    nb = E // BE
    return pl.pallas_call(
        body,
        grid=(nb,),
        in_specs=[pl.BlockSpec((BE, PF), lambda i: (i, 0))
                  for _ in range(NPASS)]
        + [pl.BlockSpec((BE, 1), lambda i: (i, 0)),
           pl.BlockSpec((2, HP), lambda i: (0, 0)),
           pl.BlockSpec((1, HP), lambda i: (0, 0)),
           pl.BlockSpec((1, HP), lambda i: (0, 0)),
           pl.BlockSpec((HP, HP), lambda i: (0, 0)),
           pl.BlockSpec((1, HP), lambda i: (0, 0))],
        out_specs=[pl.BlockSpec((BE, PF), lambda i: (i, 0))
                   for _ in range(NPASS)],
        out_shape=[jax.ShapeDtypeStruct((E, PF), jnp.float32)
                   for _ in range(NPASS)],
    )(*us, xe2, estats, g, b, W22, g2)


def _tc_final_mean(tmp2, h1, nstats, g, b):
    def body(tmp_r, h1_r, st_r, g_r, b_r, o_ref):
        i = pl.program_id(0)
        m = st_r[0:1] * (1.0 / N)
        var = st_r[1:2] * (1.0 / N) - m * m
        inv = 1.0 / jnp.sqrt(var + EPS)
        h2 = h1_r[...] + jnp.maximum(
            (tmp_r[...] - m) * inv * g_r[...] + b_r[...], 0.0)
        @pl.when(i == 0)
        def _():
            o_ref[...] = jnp.zeros((1, HP), jnp.float32)
        o_ref[...] = o_ref[...] + jnp.sum(h2, 0, keepdims=True)
        @pl.when(i == (N // BM) - 1)
        def _():
            o_ref[...] = o_ref[...] * (1.0 / N)

    nb = N // BM
    return pl.pallas_call(
        body,
        grid=(nb,),
        in_specs=[
            pl.BlockSpec((BM, HP), lambda i: (i, 0)),
            pl.BlockSpec((BM, HP), lambda i: (i, 0)),
            pl.BlockSpec((2, HP), lambda i: (0, 0)),
            pl.BlockSpec((1, HP), lambda i: (0, 0)),
            pl.BlockSpec((1, HP), lambda i: (0, 0)),
        ],
        out_specs=pl.BlockSpec((1, HP), lambda i: (0, 0)),
        out_shape=jax.ShapeDtypeStruct((1, HP), jnp.float32),
    )(tmp2, h1, nstats, g, b)


def _padw(a, shape):
    out = jnp.zeros(shape, jnp.float32)
    if a.ndim == 1:
        return out.at[:a.shape[0]].set(a)
    return out.at[:a.shape[0], :a.shape[1]].set(a)


def kernel(nodes_feat, edges_feat, nodes_num_norm_sqrt, edges_num_norm_sqrt,
           edge_index, emb_h_w, emb_h_b, emb_e_w, emb_e_b,
           l1_W, l1_b, l1_bng_h, l1_bnb_h, l1_bng_e, l1_bnb_e,
           lo_W, lo_b, lo_bng_h, lo_bnb_h, lo_bng_e, lo_bnb_e):
    xe = edges_feat.reshape(E)
    se = edges_num_norm_sqrt.reshape(E)
    xe2 = edges_feat
    sn = nodes_num_norm_sqrt
    we = emb_e_w[0]

    Wh = _padw(emb_h_w, (IN_DIM, HP))
    bh = _padw(emb_h_b, (HP,)).reshape(1, HP)
    def catW(Wl, bl):
        Wc = jnp.zeros((HP, 4 * HP), jnp.float32)
        bc = jnp.zeros((4 * HP,), jnp.float32)
        for k, idx in enumerate((0, 1, 3, 4)):
            Wc = Wc.at[:H, k * HP:k * HP + H].set(Wl[idx])
            bc = bc.at[k * HP:k * HP + H].set(bl[idx])
        return Wc, bc.reshape(1, 4 * HP)
    Wc1, bc1 = catW(l1_W, l1_b)
    Wc2, bc2 = catW(lo_W, lo_b)
    c1 = _padw(we @ l1_W[2], (HP,))
    d1 = _padw(emb_e_b @ l1_W[2] + l1_b[2], (HP,)).reshape(1, HP)
    W22 = _padw(lo_W[2], (HP, HP))
    g2 = _padw(we @ lo_W[2], (HP,)).reshape(1, HP)
    d2 = _padw(emb_e_b @ lo_W[2] + lo_b[2], (HP,)).reshape(1, HP)
    g1h = _padw(l1_bng_h, (HP,)).reshape(1, HP)
    b1h = _padw(l1_bnb_h, (HP,)).reshape(1, HP)
    g1e = _padw(l1_bng_e, (HP,)).reshape(1, HP)
    b1e = _padw(l1_bnb_e, (HP,)).reshape(1, HP)
    g2h = _padw(lo_bng_h, (HP,)).reshape(1, HP)
    b2h = _padw(lo_bnb_h, (HP,)).reshape(1, HP)

    prep = _tc_prep(nodes_feat, Wh, bh, Wc1, bc1, d1)
    h0, Ah1 = prep[0], prep[1]
    src1 = prep[2:2 + NPASS]
    dst1 = prep[2 + NPASS:2 + 2 * NPASS]

    accs1, us = [], []
    for p in range(NPASS):
        acc, u = _sc_pass_l1(edge_index, xe, se, src1[p], dst1[p],
                             lax.dynamic_slice(c1, (PF * p,), (PF,)))
        accs1.append(acc)
        us.append(u)

    tmp1, nst1 = _tc_node_gate(accs1, Ah1, sn)
    est = _tc_edge_stats(us)
    upd = _tc_node_update(tmp1, h0, nst1, g1h, b1h, Wc2, bc2, d2)
    h1, Ah2 = upd[0], upd[1]
    src2 = upd[2:2 + NPASS]
    dst2 = upd[2 + NPASS:2 + 2 * NPASS]
    ces = _tc_edge_ce(us, xe2, est, g1e, b1e, W22, g2)

    accs2 = [_sc_pass_l2(edge_index, ces[p], src2[p], dst2[p])
             for p in range(NPASS)]

    tmp2, nst2 = _tc_node_gate(accs2, Ah2, sn)
    hg = _tc_final_mean(tmp2, h1, nst2, g2h, b2h)
    return hg[:, :H]


# speedup vs baseline: 1.4349x; 1.2305x over previous
import dataclasses
import functools

import jax
import jax.numpy as jnp
from jax import lax
from jax.experimental import pallas as pl
from jax.experimental.pallas import tpu as pltpu
from jax.experimental.pallas import tpu_sc as plsc

N = 50000
E = 800000
IN_DIM = 128
H = 70
HP = 80
PF = 16
NPASS = HP // PF
SW = 2 * PF
CH = 256
SUB = CH // 128
NTILE = 32
EP = 819200
NCHUNK = EP // CH
CPT = NCHUNK // NTILE
TPAIR = CPT // 2
NSUB = 16
NP = 50048
RPT = NP // NSUB
BM = 2000
BE = 2000
EPS = 1e-5

_SC_CP = pltpu.CompilerParams()
if "needs_layout_passes" in pltpu.CompilerParams.__dataclass_fields__:
    _SC_CP = dataclasses.replace(_SC_CP, needs_layout_passes=False)
if "use_tc_tiling_on_sc" in pltpu.CompilerParams.__dataclass_fields__:
    _SC_CP = dataclasses.replace(_SC_CP, use_tc_tiling_on_sc=False)


_SC_SET = [
    pltpu.VMEM((SUB, 128), jnp.int32),
    pltpu.VMEM((SUB, 128), jnp.int32),
    pltpu.VMEM((CH, SW), jnp.float32),
    pltpu.VMEM((CH, PF), jnp.float32),
    pltpu.VMEM((CH, SW), jnp.bfloat16),
    pltpu.VMEM((CH, PF), jnp.float32),
    pltpu.VMEM((CH,), jnp.float32),
    pltpu.SemaphoreType.DMA,
    pltpu.SemaphoreType.DMA,
    pltpu.SemaphoreType.DMA,
]
_SC_SCRATCH = _SC_SET + _SC_SET + [
    pltpu.VMEM((PF,), jnp.float32),
    pltpu.VMEM_SHARED((NP, SW), jnp.bfloat16),
]


def _zero_and_barrier(outv, acc, s):
    zero32 = jnp.zeros((SW,), jnp.bfloat16)

    @pl.loop(0, CH)
    def _(j):
        outv[j, pl.ds(0, SW)] = zero32
    r0 = s * RPT
    @pl.loop(0, RPT // CH)
    def _(q):
        pltpu.sync_copy(outv, acc.at[pl.ds(r0 + q * CH, CH)])
    @pl.when(RPT % CH != 0)
    def _():
        pltpu.sync_copy(outv.at[pl.ds(0, RPT % CH)],
                        acc.at[pl.ds(r0 + (RPT // CH) * CH, RPT % CH)])
    plsc.subcore_barrier()


def _flush(acc, accum_out, s, c):
    plsc.subcore_barrier()
    r0 = s * RPT
    @pl.loop(0, RPT // CH)
    def _(q):
        pltpu.sync_copy(acc.at[pl.ds(r0 + q * CH, CH)],
                        accum_out.at[c, pl.ds(r0 + q * CH, CH)])
    @pl.when(RPT % CH != 0)
    def _():
        pltpu.sync_copy(acc.at[pl.ds(r0 + (RPT // CH) * CH, RPT % CH)],
                        accum_out.at[c, pl.ds(r0 + (RPT // CH) * CH,
                                              RPT % CH)])


def _sc_edge_pipeline(sets, acc, wid, ei, st, dt, aux, compute, fire_aux_out,
                      wait_aux_out):
    def fire_loads(S, cid):
        isrc, idst, _, _, _, _, _, semL, _, _ = sets[S]
        crow = cid * SUB
        pltpu.async_copy(ei.at[0, pl.ds(crow, SUB)], isrc, semL)
        pltpu.async_copy(ei.at[1, pl.ds(crow, SUB)], idst, semL)
        ref, hbm = aux(S)
        if hbm is not None:
            pltpu.async_copy(hbm.at[pl.ds(cid * CH, CH)], ref, semL)

    def wait_loads(S, cid):
        isrc, idst, _, _, _, _, _, semL, _, _ = sets[S]
        crow = cid * SUB
        pltpu.make_async_copy(ei.at[0, pl.ds(crow, SUB)], isrc, semL).wait()
        pltpu.make_async_copy(ei.at[1, pl.ds(crow, SUB)], idst, semL).wait()
        ref, hbm = aux(S)
        if hbm is not None:
            pltpu.make_async_copy(hbm.at[pl.ds(cid * CH, CH)], ref,
                                  semL).wait()

    def fire_gathers(S):
        isrc, idst, rsrc, rdst, _, _, _, _, semG, _ = sets[S]
        for q in range(SUB):
            pltpu.async_copy(st.at[isrc.at[q]],
                             rsrc.at[pl.ds(q * 128, 128)], semG)
            pltpu.async_copy(dt.at[idst.at[q]],
                             rdst.at[pl.ds(q * 128, 128)], semG)

    def wait_gathers(S):
        isrc, idst, rsrc, rdst, _, _, _, _, semG, _ = sets[S]
        for q in range(SUB):
            pltpu.make_async_copy(st.at[isrc.at[q]],
                                  rsrc.at[pl.ds(q * 128, 128)], semG).wait()
            pltpu.make_async_copy(dt.at[idst.at[q]],
                                  rdst.at[pl.ds(q * 128, 128)], semG).wait()

    def fire_scatter(S, cid):
        _, idst, _, _, outv, _, _, _, _, semS = sets[S]
        for q in range(SUB):
            pltpu.sync_copy(outv.at[pl.ds(q * 128, 128)],
                            acc.at[idst.at[q]], add=True)
        fire_aux_out(S, cid)

    def wait_scatter(S, cid):
        wait_aux_out(S, cid)

    fire_loads(0, wid)
    wait_loads(0, wid)
    fire_gathers(0)
    fire_loads(1, wid + NTILE)
    wait_loads(1, wid + NTILE)
    fire_gathers(1)

    @pl.loop(0, TPAIR)
    def _(t):
        for S in (0, 1):
            cid = wid + (2 * t + S) * NTILE
            nxt = cid + 2 * NTILE
            wait_gathers(S)
            @pl.when(t < TPAIR - 1)
            def _():
                fire_loads(S, nxt)
            @pl.when(t > 0)
            def _():
                wait_scatter(S, cid - 2 * NTILE)
            compute(S)
            fire_scatter(S, cid)
            @pl.when(t < TPAIR - 1)
            def _():
                wait_loads(S, nxt)
                fire_gathers(S)

    wait_scatter(0, wid + (2 * TPAIR - 2) * NTILE)
    wait_scatter(1, wid + (2 * TPAIR - 1) * NTILE)


def _sc_pass_l1(edge_index3, xe, srctab, dsttab, c1p, tok):
    mesh = plsc.VectorSubcoreMesh(core_axis_name="c", subcore_axis_name="s")

    @functools.partial(
        pl.kernel, mesh=mesh, compiler_params=_SC_CP,
        out_type=[jax.ShapeDtypeStruct((2, NP, SW), jnp.bfloat16),
                  jax.ShapeDtypeStruct((EP, PF), jnp.float32)],
        scratch_types=_SC_SCRATCH)
    def k(ei, xeh, st, dt, c1h, tok_r, accum_out, u_out, *scr):
        sets = [scr[0:10], scr[10:20]]
        c1v, acc = scr[20], scr[21]
        s = lax.axis_index("s")
        c = lax.axis_index("c")
        wid = s * 2 + c

        _zero_and_barrier(sets[0][4], acc, s)
        pltpu.sync_copy(c1h, c1v)
        c1 = c1v[...]

        def compute(S):
            _, _, rsrc, rdst, outv, uv, xev, _, _, _ = sets[S]

            @pl.loop(0, CH)
            def _(j):
                bh = rsrc[j, pl.ds(0, PF)]
                dh = rsrc[j, pl.ds(PF, PF)]
                eh = rdst[j, pl.ds(0, PF)]
                xv = plsc.load_gather(xev, [jnp.full((PF,), j, jnp.int32)])
                enew = xv * c1 + dh + eh
                sig = 1.0 / (1.0 + jnp.exp(-enew))
                uv[j, pl.ds(0, PF)] = enew
                outv[j, pl.ds(0, SW)] = plsc.pack(
                    sig * bh, sig, format=plsc.PackFormat.INTERLEAVED)

        def fire_u(S, cid):
            uv = sets[S][5]
            pltpu.sync_copy(uv, u_out.at[pl.ds(cid * CH, CH)])

        def wait_u(S, cid):
            pass

        _sc_edge_pipeline(sets, acc, wid, ei, st, dt,
                          lambda S: (sets[S][6], xeh), compute,
                          fire_u, wait_u)
        _flush(acc, accum_out, s, c)

    return k(edge_index3, xe, srctab, dsttab, c1p, tok)


def _sc_pass_l2(edge_index3, ce, srctab, dsttab, tok):
    mesh = plsc.VectorSubcoreMesh(core_axis_name="c", subcore_axis_name="s")

    @functools.partial(
        pl.kernel, mesh=mesh, compiler_params=_SC_CP,
        out_type=jax.ShapeDtypeStruct((2, NP, SW), jnp.bfloat16),
        scratch_types=_SC_SCRATCH)
    def k(ei, ceh, st, dt, tok_r, accum_out, *scr):
        sets = [scr[0:10], scr[10:20]]
        acc = scr[21]
        s = lax.axis_index("s")
        c = lax.axis_index("c")
        wid = s * 2 + c

        _zero_and_barrier(sets[0][4], acc, s)

        def compute(S):
            _, _, rsrc, rdst, outv, cev, _, _, _, _ = sets[S]

            @pl.loop(0, CH)
            def _(j):
                bh = rsrc[j, pl.ds(0, PF)]
                dh = rsrc[j, pl.ds(PF, PF)]
                eh = rdst[j, pl.ds(0, PF)]
                cv = cev[j, pl.ds(0, PF)]
                enew = cv + dh + eh
                sig = 1.0 / (1.0 + jnp.exp(-enew))
                outv[j, pl.ds(0, SW)] = plsc.pack(
                    sig * bh, sig, format=plsc.PackFormat.INTERLEAVED)

        def noop(S, cid):
            pass

        _sc_edge_pipeline(sets, acc, wid, ei, st, dt,
                          lambda S: (sets[S][5], ceh), compute,
                          noop, noop)
        _flush(acc, accum_out, s, c)

    return k(edge_index3, ce, srctab, dsttab, tok)


def _tc_prep(xn, Wh, bh, Wc, bc, d1):
    def body(xn_r, Wh_r, bh_r, Wc_r, bc_r, d1_r, h0_o, Ah_o, *tab_o):
        h0 = jnp.dot(xn_r[...], Wh_r[...],
                     preferred_element_type=jnp.float32) + bh_r[...]
        proj = jnp.dot(h0, Wc_r[...],
                       preferred_element_type=jnp.float32) + bc_r[...]
        h0_o[...] = h0
        Ah_o[...] = proj[:, 0:HP]
        Bh = proj[:, HP:2 * HP]
        Dh = proj[:, 2 * HP:3 * HP]
        Eh = proj[:, 3 * HP:4 * HP]
        for p in range(NPASS):
            sl = slice(PF * p, PF * p + PF)
            tab_o[p][...] = jnp.concatenate([Bh[:, sl], Dh[:, sl]], axis=1)
            tab_o[NPASS + p][...] = Eh[:, sl] + d1_r[:, sl]

    nb = N // BM
    return pl.pallas_call(
        body,
        grid=(nb,),
        in_specs=[
            pl.BlockSpec((BM, IN_DIM), lambda i: (i, 0)),
            pl.BlockSpec((IN_DIM, HP), lambda i: (0, 0)),
            pl.BlockSpec((1, HP), lambda i: (0, 0)),
            pl.BlockSpec((HP, 4 * HP), lambda i: (0, 0)),
            pl.BlockSpec((1, 4 * HP), lambda i: (0, 0)),
            pl.BlockSpec((1, HP), lambda i: (0, 0)),
        ],
        out_specs=[pl.BlockSpec((BM, HP), lambda i: (i, 0)),
                   pl.BlockSpec((BM, HP), lambda i: (i, 0))]
        + [pl.BlockSpec((BM, SW), lambda i: (i, 0)) for _ in range(NPASS)]
        + [pl.BlockSpec((BM, PF), lambda i: (i, 0)) for _ in range(NPASS)],
        out_shape=[jax.ShapeDtypeStruct((N, HP), jnp.float32),
                   jax.ShapeDtypeStruct((N, HP), jnp.float32)]
        + [jax.ShapeDtypeStruct((N, SW), jnp.float32) for _ in range(NPASS)]
        + [jax.ShapeDtypeStruct((N, PF), jnp.float32) for _ in range(NPASS)],
    )(xn, Wh, bh, Wc, bc, d1)


def _unpack_lo(w):
    return jax.lax.bitcast_convert_type(jnp.left_shift(w, 16), jnp.float32)


def _unpack_hi(w):
    return jax.lax.bitcast_convert_type(
        jnp.bitwise_and(w, jnp.int32(-65536)), jnp.float32)


def _tc_node_gate(accums, Ah, sn):
    def body(*refs):
        a5 = refs[:NPASS]
        Ah_r, sn_r, tmp_o, st_o = refs[NPASS:]
        i = pl.program_id(0)
        num = jnp.concatenate(
            [_unpack_lo(a[0]) + _unpack_lo(a[1]) for a in a5], axis=1)
        den = jnp.concatenate(
            [_unpack_hi(a[0]) + _unpack_hi(a[1]) for a in a5], axis=1)
        tmp = (Ah_r[...] + num / (den + 1e-6)) * sn_r[...]
        tmp_o[...] = tmp
        @pl.when(i == 0)
        def _():
            st_o[...] = jnp.zeros((2, HP), jnp.float32)
        st_o[...] = st_o[...] + jnp.concatenate(
            [jnp.sum(tmp, 0, keepdims=True),
             jnp.sum(tmp * tmp, 0, keepdims=True)], axis=0)

    nb = N // BM
    return pl.pallas_call(
        body,
        grid=(nb,),
        in_specs=[pl.BlockSpec((2, BM, PF), lambda i: (0, i, 0))
                  for _ in range(NPASS)]
        + [pl.BlockSpec((BM, HP), lambda i: (i, 0)),
           pl.BlockSpec((BM, 1), lambda i: (i, 0))],
        out_specs=[pl.BlockSpec((BM, HP), lambda i: (i, 0)),
                   pl.BlockSpec((2, HP), lambda i: (0, 0))],
        out_shape=[jax.ShapeDtypeStruct((N, HP), jnp.float32),
                   jax.ShapeDtypeStruct((2, HP), jnp.float32)],
    )(*accums, Ah, sn)


def _tc_edge_stats(us, se2):
    def body(*refs):
        u5 = refs[:NPASS]
        se_r, st_o = refs[NPASS:]
        i = pl.program_id(0)
        u = jnp.concatenate([r[...] for r in u5], axis=1) * se_r[...]
        @pl.when(i == 0)
        def _():
            st_o[...] = jnp.zeros((2, HP), jnp.float32)
        st_o[...] = st_o[...] + jnp.concatenate(
            [jnp.sum(u, 0, keepdims=True),
             jnp.sum(u * u, 0, keepdims=True)], axis=0)

    nb = E // BE
    return pl.pallas_call(
        body,
        grid=(nb,),
        in_specs=[pl.BlockSpec((BE, PF), lambda i: (i, 0))
                  for _ in range(NPASS)]
        + [pl.BlockSpec((BE, 1), lambda i: (i, 0))],
        out_specs=pl.BlockSpec((2, HP), lambda i: (0, 0)),
        out_shape=jax.ShapeDtypeStruct((2, HP), jnp.float32),
    )(*us, se2)


def _tc_node_update(tmp, h0, nstats, g, b, Wc, bc, d2):
    def body(tmp_r, h0_r, st_r, g_r, b_r, Wc_r, bc_r, d2_r,
             h1_o, Ah_o, *tab_o):
        m = st_r[0:1] * (1.0 / N)
        var = st_r[1:2] * (1.0 / N) - m * m
        inv = 1.0 / jnp.sqrt(var + EPS)
        h1 = h0_r[...] + jnp.maximum(
            (tmp_r[...] - m) * inv * g_r[...] + b_r[...], 0.0)
        proj = jnp.dot(h1, Wc_r[...],
                       preferred_element_type=jnp.float32) + bc_r[...]
        h1_o[...] = h1
        Ah_o[...] = proj[:, 0:HP]
        Bh = proj[:, HP:2 * HP]
        Dh = proj[:, 2 * HP:3 * HP]
        Eh = proj[:, 3 * HP:4 * HP]
        for p in range(NPASS):
            sl = slice(PF * p, PF * p + PF)
            tab_o[p][...] = jnp.concatenate([Bh[:, sl], Dh[:, sl]], axis=1)
            tab_o[NPASS + p][...] = Eh[:, sl] + d2_r[:, sl]

    nb = N // BM
    return pl.pallas_call(
        body,
        grid=(nb,),
        in_specs=[
            pl.BlockSpec((BM, HP), lambda i: (i, 0)),
            pl.BlockSpec((BM, HP), lambda i: (i, 0)),
            pl.BlockSpec((2, HP), lambda i: (0, 0)),
            pl.BlockSpec((1, HP), lambda i: (0, 0)),
            pl.BlockSpec((1, HP), lambda i: (0, 0)),
            pl.BlockSpec((HP, 4 * HP), lambda i: (0, 0)),
            pl.BlockSpec((1, 4 * HP), lambda i: (0, 0)),
            pl.BlockSpec((1, HP), lambda i: (0, 0)),
        ],
        out_specs=[pl.BlockSpec((BM, HP), lambda i: (i, 0)),
                   pl.BlockSpec((BM, HP), lambda i: (i, 0))]
        + [pl.BlockSpec((BM, SW), lambda i: (i, 0)) for _ in range(NPASS)]
        + [pl.BlockSpec((BM, PF), lambda i: (i, 0)) for _ in range(NPASS)],
        out_shape=[jax.ShapeDtypeStruct((N, HP), jnp.float32),
                   jax.ShapeDtypeStruct((N, HP), jnp.float32)]
        + [jax.ShapeDtypeStruct((N, SW), jnp.float32) for _ in range(NPASS)]
        + [jax.ShapeDtypeStruct((N, PF), jnp.float32) for _ in range(NPASS)],
    )(tmp, h0, nstats, g, b, Wc, bc, d2)


def _tc_edge_ce(us, xe2, se2, estats, g, b, W22, g2):
    def body(*refs):
        u5 = refs[:NPASS]
        xe_r, se_r, st_r, g_r, b_r, W_r, g2_r = refs[NPASS:NPASS + 7]
        ce_o = refs[NPASS + 7:]
        u = jnp.concatenate([r[...] for r in u5], axis=1) * se_r[...]
        m = st_r[0:1] * (1.0 / E)
        var = st_r[1:2] * (1.0 / E) - m * m
        inv = 1.0 / jnp.sqrt(var + EPS)
        r = jnp.maximum((u - m) * inv * g_r[...] + b_r[...], 0.0)
        ce = jnp.dot(r, W_r[...],
                     preferred_element_type=jnp.float32) + xe_r[...] * g2_r[...]
        for p in range(NPASS):
            ce_o[p][...] = ce[:, PF * p:PF * p + PF]

    nb = E // BE
    return pl.pallas_call(
        body,
        grid=(nb,),
        in_specs=[pl.BlockSpec((BE, PF), lambda i: (i, 0))
                  for _ in range(NPASS)]
        + [pl.BlockSpec((BE, 1), lambda i: (i, 0)),
           pl.BlockSpec((BE, 1), lambda i: (i, 0)),
           pl.BlockSpec((2, HP), lambda i: (0, 0)),
           pl.BlockSpec((1, HP), lambda i: (0, 0)),
           pl.BlockSpec((1, HP), lambda i: (0, 0)),
           pl.BlockSpec((HP, HP), lambda i: (0, 0)),
           pl.BlockSpec((1, HP), lambda i: (0, 0))],
        out_specs=[pl.BlockSpec((BE, PF), lambda i: (i, 0))
                   for _ in range(NPASS)],
        out_shape=[jax.ShapeDtypeStruct((EP, PF), jnp.float32)
                   for _ in range(NPASS)],
    )(*us, xe2, se2, estats, g, b, W22, g2)


def _tc_final_mean(tmp2, h1, nstats, g, b):
    def body(tmp_r, h1_r, st_r, g_r, b_r, o_ref):
        i = pl.program_id(0)
        m = st_r[0:1] * (1.0 / N)
        var = st_r[1:2] * (1.0 / N) - m * m
        inv = 1.0 / jnp.sqrt(var + EPS)
        h2 = h1_r[...] + jnp.maximum(
            (tmp_r[...] - m) * inv * g_r[...] + b_r[...], 0.0)
        @pl.when(i == 0)
        def _():
            o_ref[...] = jnp.zeros((1, HP), jnp.float32)
        o_ref[...] = o_ref[...] + jnp.sum(h2, 0, keepdims=True)
        @pl.when(i == (N // BM) - 1)
        def _():
            o_ref[...] = o_ref[...] * (1.0 / N)

    nb = N // BM
    return pl.pallas_call(
        body,
        grid=(nb,),
        in_specs=[
            pl.BlockSpec((BM, HP), lambda i: (i, 0)),
            pl.BlockSpec((BM, HP), lambda i: (i, 0)),
            pl.BlockSpec((2, HP), lambda i: (0, 0)),
            pl.BlockSpec((1, HP), lambda i: (0, 0)),
            pl.BlockSpec((1, HP), lambda i: (0, 0)),
        ],
        out_specs=pl.BlockSpec((1, HP), lambda i: (0, 0)),
        out_shape=jax.ShapeDtypeStruct((1, HP), jnp.float32),
    )(tmp2, h1, nstats, g, b)


def _pairs_i32(acc_bf):
    return jax.lax.bitcast_convert_type(
        acc_bf.reshape(2, NP, PF, 2), jnp.int32)


def _padw(a, shape):
    out = jnp.zeros(shape, jnp.float32)
    if a.ndim == 1:
        return out.at[:a.shape[0]].set(a)
    return out.at[:a.shape[0], :a.shape[1]].set(a)


def kernel(nodes_feat, edges_feat, nodes_num_norm_sqrt, edges_num_norm_sqrt,
           edge_index, emb_h_w, emb_h_b, emb_e_w, emb_e_b,
           l1_W, l1_b, l1_bng_h, l1_bnb_h, l1_bng_e, l1_bnb_e,
           lo_W, lo_b, lo_bng_h, lo_bnb_h, lo_bng_e, lo_bnb_e):
    xe2 = edges_feat
    se2 = edges_num_norm_sqrt
    sn = nodes_num_norm_sqrt
    we = emb_e_w[0]

    xep = jnp.zeros((EP,), jnp.float32).at[:E].set(edges_feat.reshape(E))
    eip = jnp.full((2, EP), N, jnp.int32).at[:, :E].set(edge_index)
    eip = eip.at[0, E:].set(0)
    ei3 = eip.reshape(2, EP // 128, 128)

    Wh = _padw(emb_h_w, (IN_DIM, HP))
    bh = _padw(emb_h_b, (HP,)).reshape(1, HP)
    def catW(Wl, bl):
        Wc = jnp.zeros((HP, 4 * HP), jnp.float32)
        bc = jnp.zeros((4 * HP,), jnp.float32)
        for k, idx in enumerate((0, 1, 3, 4)):
            Wc = Wc.at[:H, k * HP:k * HP + H].set(Wl[idx])
            bc = bc.at[k * HP:k * HP + H].set(bl[idx])
        return Wc, bc.reshape(1, 4 * HP)
    Wc1, bc1 = catW(l1_W, l1_b)
    Wc2, bc2 = catW(lo_W, lo_b)
    c1 = _padw(we @ l1_W[2], (HP,))
    d1 = _padw(emb_e_b @ l1_W[2] + l1_b[2], (HP,)).reshape(1, HP)
    W22 = _padw(lo_W[2], (HP, HP))
    g2 = _padw(we @ lo_W[2], (HP,)).reshape(1, HP)
    d2 = _padw(emb_e_b @ lo_W[2] + lo_b[2], (HP,)).reshape(1, HP)
    g1h = _padw(l1_bng_h, (HP,)).reshape(1, HP)
    b1h = _padw(l1_bnb_h, (HP,)).reshape(1, HP)
    g1e = _padw(l1_bng_e, (HP,)).reshape(1, HP)
    b1e = _padw(l1_bnb_e, (HP,)).reshape(1, HP)
    g2h = _padw(lo_bng_h, (HP,)).reshape(1, HP)
    b2h = _padw(lo_bnb_h, (HP,)).reshape(1, HP)

    prep = _tc_prep(nodes_feat, Wh, bh, Wc1, bc1, d1)
    h0, Ah1 = prep[0], prep[1]
    src1 = prep[2:2 + NPASS]
    dst1 = prep[2 + NPASS:2 + 2 * NPASS]

    accs1, us = [], []
    tok = h0
    for p in range(NPASS):
        acc, u = _sc_pass_l1(ei3, xep, src1[p], dst1[p],
                             lax.dynamic_slice(c1, (PF * p,), (PF,)), tok)
        accs1.append(acc)
        us.append(u)
        tok = acc

    accs1 = [_pairs_i32(a) for a in accs1]
    tmp1, nst1 = _tc_node_gate(accs1, Ah1, sn)
    est = _tc_edge_stats(us, se2)
    upd = _tc_node_update(tmp1, h0, nst1, g1h, b1h, Wc2, bc2, d2)
    h1, Ah2 = upd[0], upd[1]
    src2 = upd[2:2 + NPASS]
    dst2 = upd[2 + NPASS:2 + 2 * NPASS]
    ces = _tc_edge_ce(us, xe2, se2, est, g1e, b1e, W22, g2)

    accs2 = []
    tok = tmp1
    for p in range(NPASS):
        acc = _sc_pass_l2(ei3, ces[p], src2[p], dst2[p], tok)
        accs2.append(acc)
        tok = acc

    accs2 = [_pairs_i32(a) for a in accs2]
    tmp2, nst2 = _tc_node_gate(accs2, Ah2, sn)
    hg = _tc_final_mean(tmp2, h1, nst2, g2h, b2h)
    return hg[:, :H]


# speedup vs baseline: 1.9841x; 1.3828x over previous
import dataclasses
import functools

import jax
import jax.numpy as jnp
from jax import lax
from jax.experimental import pallas as pl
from jax.experimental.pallas import tpu as pltpu
from jax.experimental.pallas import tpu_sc as plsc

N = 50000
E = 800000
IN_DIM = 128
H = 70
HP = 80
PF = 16
NPASS = HP // PF
SW = 2 * PF
CH = 256
SUB = CH // 128
NTILE = 32
EP = 819200
NCHUNK = EP // CH
CPT = NCHUNK // NTILE
TPAIR = CPT // 2
NSUB = 16
NP = 50048
RPT = NP // NSUB
BM = 2000
BE = 2000
EPS = 1e-5

_SC_CP = pltpu.CompilerParams()
if "needs_layout_passes" in pltpu.CompilerParams.__dataclass_fields__:
    _SC_CP = dataclasses.replace(_SC_CP, needs_layout_passes=False)
if "use_tc_tiling_on_sc" in pltpu.CompilerParams.__dataclass_fields__:
    _SC_CP = dataclasses.replace(_SC_CP, use_tc_tiling_on_sc=False)


_SC_SET = [
    pltpu.VMEM((SUB, 128), jnp.int32),
    pltpu.VMEM((SUB, 128), jnp.int32),
    pltpu.VMEM((CH, SW), jnp.float32),
    pltpu.VMEM((CH, PF), jnp.float32),
    pltpu.VMEM((CH, SW), jnp.bfloat16),
    pltpu.VMEM((CH, PF), jnp.float32),
    pltpu.VMEM((CH,), jnp.float32),
    pltpu.SemaphoreType.DMA,
    pltpu.SemaphoreType.DMA,
    pltpu.SemaphoreType.DMA,
]
_SC_SCRATCH = _SC_SET + _SC_SET + [
    pltpu.VMEM((PF,), jnp.float32),
    pltpu.VMEM_SHARED((NP, SW), jnp.bfloat16),
]


def _zero_and_barrier(outv, acc, s):
    zero32 = jnp.zeros((SW,), jnp.bfloat16)

    @pl.loop(0, CH)
    def _(j):
        outv[j, pl.ds(0, SW)] = zero32
    r0 = s * RPT
    @pl.loop(0, RPT // CH)
    def _(q):
        pltpu.sync_copy(outv, acc.at[pl.ds(r0 + q * CH, CH)])
    @pl.when(RPT % CH != 0)
    def _():
        pltpu.sync_copy(outv.at[pl.ds(0, RPT % CH)],
                        acc.at[pl.ds(r0 + (RPT // CH) * CH, RPT % CH)])
    plsc.subcore_barrier()


def _flush(acc, accum_out, s, c):
    plsc.subcore_barrier()
    r0 = s * RPT
    @pl.loop(0, RPT // CH)
    def _(q):
        pltpu.sync_copy(acc.at[pl.ds(r0 + q * CH, CH)],
                        accum_out.at[c, pl.ds(r0 + q * CH, CH)])
    @pl.when(RPT % CH != 0)
    def _():
        pltpu.sync_copy(acc.at[pl.ds(r0 + (RPT // CH) * CH, RPT % CH)],
                        accum_out.at[c, pl.ds(r0 + (RPT // CH) * CH,
                                              RPT % CH)])


def _sc_edge_pipeline(sets, acc, wid, ei, st, dt, aux, compute, fire_aux_out,
                      wait_aux_out):
    def fire_loads(S, cid):
        isrc, idst, _, _, _, _, _, semL, _, _ = sets[S]
        crow = cid * SUB
        pltpu.async_copy(ei.at[0, pl.ds(crow, SUB)], isrc, semL)
        pltpu.async_copy(ei.at[1, pl.ds(crow, SUB)], idst, semL)
        ref, hbm = aux(S)
        if hbm is not None:
            pltpu.async_copy(hbm.at[pl.ds(cid * CH, CH)], ref, semL)

    def wait_loads(S, cid):
        isrc, idst, _, _, _, _, _, semL, _, _ = sets[S]
        crow = cid * SUB
        pltpu.make_async_copy(ei.at[0, pl.ds(crow, SUB)], isrc, semL).wait()
        pltpu.make_async_copy(ei.at[1, pl.ds(crow, SUB)], idst, semL).wait()
        ref, hbm = aux(S)
        if hbm is not None:
            pltpu.make_async_copy(hbm.at[pl.ds(cid * CH, CH)], ref,
                                  semL).wait()

    def fire_gathers(S):
        isrc, idst, rsrc, rdst, _, _, _, _, semG, _ = sets[S]
        for q in range(SUB):
            pltpu.async_copy(st.at[isrc.at[q]],
                             rsrc.at[pl.ds(q * 128, 128)], semG)
            pltpu.async_copy(dt.at[idst.at[q]],
                             rdst.at[pl.ds(q * 128, 128)], semG)

    def wait_gathers(S):
        isrc, idst, rsrc, rdst, _, _, _, _, semG, _ = sets[S]
        for q in range(SUB):
            pltpu.make_async_copy(st.at[isrc.at[q]],
                                  rsrc.at[pl.ds(q * 128, 128)], semG).wait()
            pltpu.make_async_copy(dt.at[idst.at[q]],
                                  rdst.at[pl.ds(q * 128, 128)], semG).wait()

    def fire_scatter(S, cid):
        _, idst, _, _, outv, _, _, _, _, semS = sets[S]
        for q in range(SUB):
            pltpu.sync_copy(outv.at[pl.ds(q * 128, 128)],
                            acc.at[idst.at[q]], add=True)
        fire_aux_out(S, cid)

    def wait_scatter(S, cid):
        wait_aux_out(S, cid)

    fire_loads(0, wid)
    wait_loads(0, wid)
    fire_gathers(0)
    fire_loads(1, wid + NTILE)
    wait_loads(1, wid + NTILE)
    fire_gathers(1)

    @pl.loop(0, TPAIR)
    def _(t):
        for S in (0, 1):
            cid = wid + (2 * t + S) * NTILE
            nxt = cid + 2 * NTILE
            wait_gathers(S)
            @pl.when(t < TPAIR - 1)
            def _():
                fire_loads(S, nxt)
            @pl.when(t > 0)
            def _():
                wait_scatter(S, cid - 2 * NTILE)
            compute(S)
            fire_scatter(S, cid)
            @pl.when(t < TPAIR - 1)
            def _():
                wait_loads(S, nxt)
                fire_gathers(S)

    wait_scatter(0, wid + (2 * TPAIR - 2) * NTILE)
    wait_scatter(1, wid + (2 * TPAIR - 1) * NTILE)


def _sc_pass_l1(edge_index3, xe, srctab, dsttab, c1p, tok):
    mesh = plsc.VectorSubcoreMesh(core_axis_name="c", subcore_axis_name="s")

    @functools.partial(
        pl.kernel, mesh=mesh, compiler_params=_SC_CP,
        out_type=[jax.ShapeDtypeStruct((2, NP, SW), jnp.bfloat16),
                  jax.ShapeDtypeStruct((EP, PF), jnp.float32)],
        scratch_types=_SC_SCRATCH)
    def k(ei, xeh, st, dt, c1h, tok_r, accum_out, u_out, *scr):
        sets = [scr[0:10], scr[10:20]]
        c1v, acc = scr[20], scr[21]
        s = lax.axis_index("s")
        c = lax.axis_index("c")
        wid = s * 2 + c

        _zero_and_barrier(sets[0][4], acc, s)
        pltpu.sync_copy(c1h, c1v)
        c1 = c1v[...]

        def compute(S):
            _, _, rsrc, rdst, outv, uv, xev, _, _, _ = sets[S]

            @plsc.parallel_loop(0, CH, 1, unroll=4)
            def _(j):
                bh = rsrc[j, pl.ds(0, PF)]
                dh = rsrc[j, pl.ds(PF, PF)]
                eh = rdst[j, pl.ds(0, PF)]
                xv = plsc.load_gather(xev, [jnp.full((PF,), j, jnp.int32)])
                enew = xv * c1 + dh + eh
                sig = 1.0 / (1.0 + jnp.exp(-enew))
                uv[j, pl.ds(0, PF)] = enew
                outv[j, pl.ds(0, SW)] = plsc.pack(
                    sig * bh, sig, format=plsc.PackFormat.INTERLEAVED)

        def fire_u(S, cid):
            uv, semS = sets[S][5], sets[S][9]
            pltpu.async_copy(uv, u_out.at[pl.ds(cid * CH, CH)], semS)

        def wait_u(S, cid):
            uv, semS = sets[S][5], sets[S][9]
            pltpu.make_async_copy(uv, u_out.at[pl.ds(cid * CH, CH)],
                                  semS).wait()

        _sc_edge_pipeline(sets, acc, wid, ei, st, dt,
                          lambda S: (sets[S][6], xeh), compute,
                          fire_u, wait_u)
        _flush(acc, accum_out, s, c)

    return k(edge_index3, xe, srctab, dsttab, c1p, tok)


def _sc_pass_l2(edge_index3, ce, srctab, dsttab, tok):
    mesh = plsc.VectorSubcoreMesh(core_axis_name="c", subcore_axis_name="s")

    @functools.partial(
        pl.kernel, mesh=mesh, compiler_params=_SC_CP,
        out_type=jax.ShapeDtypeStruct((2, NP, SW), jnp.bfloat16),
        scratch_types=_SC_SCRATCH)
    def k(ei, ceh, st, dt, tok_r, accum_out, *scr):
        sets = [scr[0:10], scr[10:20]]
        acc = scr[21]
        s = lax.axis_index("s")
        c = lax.axis_index("c")
        wid = s * 2 + c

        _zero_and_barrier(sets[0][4], acc, s)

        def compute(S):
            _, _, rsrc, rdst, outv, cev, _, _, _, _ = sets[S]

            @plsc.parallel_loop(0, CH, 1, unroll=4)
            def _(j):
                bh = rsrc[j, pl.ds(0, PF)]
                dh = rsrc[j, pl.ds(PF, PF)]
                eh = rdst[j, pl.ds(0, PF)]
                cv = cev[j, pl.ds(0, PF)]
                enew = cv + dh + eh
                sig = 1.0 / (1.0 + jnp.exp(-enew))
                outv[j, pl.ds(0, SW)] = plsc.pack(
                    sig * bh, sig, format=plsc.PackFormat.INTERLEAVED)

        def noop(S, cid):
            pass

        _sc_edge_pipeline(sets, acc, wid, ei, st, dt,
                          lambda S: (sets[S][5], ceh), compute,
                          noop, noop)
        _flush(acc, accum_out, s, c)

    return k(edge_index3, ce, srctab, dsttab, tok)


def _tc_prep(xn, Wh, bh, Wc, bc, d1):
    def body(xn_r, Wh_r, bh_r, Wc_r, bc_r, d1_r, h0_o, Ah_o, *tab_o):
        h0 = jnp.dot(xn_r[...], Wh_r[...],
                     preferred_element_type=jnp.float32) + bh_r[...]
        proj = jnp.dot(h0, Wc_r[...],
                       preferred_element_type=jnp.float32) + bc_r[...]
        h0_o[...] = h0
        Ah_o[...] = proj[:, 0:HP]
        Bh = proj[:, HP:2 * HP]
        Dh = proj[:, 2 * HP:3 * HP]
        Eh = proj[:, 3 * HP:4 * HP]
        for p in range(NPASS):
            sl = slice(PF * p, PF * p + PF)
            tab_o[p][...] = jnp.concatenate([Bh[:, sl], Dh[:, sl]], axis=1)
            tab_o[NPASS + p][...] = Eh[:, sl] + d1_r[:, sl]

    nb = N // BM
    return pl.pallas_call(
        body,
        grid=(nb,),
        in_specs=[
            pl.BlockSpec((BM, IN_DIM), lambda i: (i, 0)),
            pl.BlockSpec((IN_DIM, HP), lambda i: (0, 0)),
            pl.BlockSpec((1, HP), lambda i: (0, 0)),
            pl.BlockSpec((HP, 4 * HP), lambda i: (0, 0)),
            pl.BlockSpec((1, 4 * HP), lambda i: (0, 0)),
            pl.BlockSpec((1, HP), lambda i: (0, 0)),
        ],
        out_specs=[pl.BlockSpec((BM, HP), lambda i: (i, 0)),
                   pl.BlockSpec((BM, HP), lambda i: (i, 0))]
        + [pl.BlockSpec((BM, SW), lambda i: (i, 0)) for _ in range(NPASS)]
        + [pl.BlockSpec((BM, PF), lambda i: (i, 0)) for _ in range(NPASS)],
        out_shape=[jax.ShapeDtypeStruct((N, HP), jnp.float32),
                   jax.ShapeDtypeStruct((N, HP), jnp.float32)]
        + [jax.ShapeDtypeStruct((N, SW), jnp.float32) for _ in range(NPASS)]
        + [jax.ShapeDtypeStruct((N, PF), jnp.float32) for _ in range(NPASS)],
    )(xn, Wh, bh, Wc, bc, d1)


def _unpack_lo(w):
    return jax.lax.bitcast_convert_type(jnp.left_shift(w, 16), jnp.float32)


def _unpack_hi(w):
    return jax.lax.bitcast_convert_type(
        jnp.bitwise_and(w, jnp.int32(-65536)), jnp.float32)


def _tc_node_gate(accums, Ah, sn):
    def body(*refs):
        a5 = refs[:NPASS]
        Ah_r, sn_r, tmp_o, st_o = refs[NPASS:]
        i = pl.program_id(0)
        num = jnp.concatenate(
            [_unpack_lo(a[0]) + _unpack_lo(a[1]) for a in a5], axis=1)
        den = jnp.concatenate(
            [_unpack_hi(a[0]) + _unpack_hi(a[1]) for a in a5], axis=1)
        tmp = (Ah_r[...] + num / (den + 1e-6)) * sn_r[...]
        tmp_o[...] = tmp
        @pl.when(i == 0)
        def _():
            st_o[...] = jnp.zeros((2, HP), jnp.float32)
        st_o[...] = st_o[...] + jnp.concatenate(
            [jnp.sum(tmp, 0, keepdims=True),
             jnp.sum(tmp * tmp, 0, keepdims=True)], axis=0)

    nb = N // BM
    return pl.pallas_call(
        body,
        grid=(nb,),
        in_specs=[pl.BlockSpec((2, BM, PF), lambda i: (0, i, 0))
                  for _ in range(NPASS)]
        + [pl.BlockSpec((BM, HP), lambda i: (i, 0)),
           pl.BlockSpec((BM, 1), lambda i: (i, 0))],
        out_specs=[pl.BlockSpec((BM, HP), lambda i: (i, 0)),
                   pl.BlockSpec((2, HP), lambda i: (0, 0))],
        out_shape=[jax.ShapeDtypeStruct((N, HP), jnp.float32),
                   jax.ShapeDtypeStruct((2, HP), jnp.float32)],
    )(*accums, Ah, sn)


def _tc_edge_stats(us, se2):
    def body(*refs):
        u5 = refs[:NPASS]
        se_r, st_o = refs[NPASS:]
        i = pl.program_id(0)
        u = jnp.concatenate([r[...] for r in u5], axis=1) * se_r[...]
        @pl.when(i == 0)
        def _():
            st_o[...] = jnp.zeros((2, HP), jnp.float32)
        st_o[...] = st_o[...] + jnp.concatenate(
            [jnp.sum(u, 0, keepdims=True),
             jnp.sum(u * u, 0, keepdims=True)], axis=0)

    nb = E // BE
    return pl.pallas_call(
        body,
        grid=(nb,),
        in_specs=[pl.BlockSpec((BE, PF), lambda i: (i, 0))
                  for _ in range(NPASS)]
        + [pl.BlockSpec((BE, 1), lambda i: (i, 0))],
        out_specs=pl.BlockSpec((2, HP), lambda i: (0, 0)),
        out_shape=jax.ShapeDtypeStruct((2, HP), jnp.float32),
    )(*us, se2)


def _tc_node_update(tmp, h0, nstats, g, b, Wc, bc, d2):
    def body(tmp_r, h0_r, st_r, g_r, b_r, Wc_r, bc_r, d2_r,
             h1_o, Ah_o, *tab_o):
        m = st_r[0:1] * (1.0 / N)
        var = st_r[1:2] * (1.0 / N) - m * m
        inv = 1.0 / jnp.sqrt(var + EPS)
        h1 = h0_r[...] + jnp.maximum(
            (tmp_r[...] - m) * inv * g_r[...] + b_r[...], 0.0)
        proj = jnp.dot(h1, Wc_r[...],
                       preferred_element_type=jnp.float32) + bc_r[...]
        h1_o[...] = h1
        Ah_o[...] = proj[:, 0:HP]
        Bh = proj[:, HP:2 * HP]
        Dh = proj[:, 2 * HP:3 * HP]
        Eh = proj[:, 3 * HP:4 * HP]
        for p in range(NPASS):
            sl = slice(PF * p, PF * p + PF)
            tab_o[p][...] = jnp.concatenate([Bh[:, sl], Dh[:, sl]], axis=1)
            tab_o[NPASS + p][...] = Eh[:, sl] + d2_r[:, sl]

    nb = N // BM
    return pl.pallas_call(
        body,
        grid=(nb,),
        in_specs=[
            pl.BlockSpec((BM, HP), lambda i: (i, 0)),
            pl.BlockSpec((BM, HP), lambda i: (i, 0)),
            pl.BlockSpec((2, HP), lambda i: (0, 0)),
            pl.BlockSpec((1, HP), lambda i: (0, 0)),
            pl.BlockSpec((1, HP), lambda i: (0, 0)),
            pl.BlockSpec((HP, 4 * HP), lambda i: (0, 0)),
            pl.BlockSpec((1, 4 * HP), lambda i: (0, 0)),
            pl.BlockSpec((1, HP), lambda i: (0, 0)),
        ],
        out_specs=[pl.BlockSpec((BM, HP), lambda i: (i, 0)),
                   pl.BlockSpec((BM, HP), lambda i: (i, 0))]
        + [pl.BlockSpec((BM, SW), lambda i: (i, 0)) for _ in range(NPASS)]
        + [pl.BlockSpec((BM, PF), lambda i: (i, 0)) for _ in range(NPASS)],
        out_shape=[jax.ShapeDtypeStruct((N, HP), jnp.float32),
                   jax.ShapeDtypeStruct((N, HP), jnp.float32)]
        + [jax.ShapeDtypeStruct((N, SW), jnp.float32) for _ in range(NPASS)]
        + [jax.ShapeDtypeStruct((N, PF), jnp.float32) for _ in range(NPASS)],
    )(tmp, h0, nstats, g, b, Wc, bc, d2)


def _tc_edge_ce(us, xe2, se2, estats, g, b, W22, g2):
    def body(*refs):
        u5 = refs[:NPASS]
        xe_r, se_r, st_r, g_r, b_r, W_r, g2_r = refs[NPASS:NPASS + 7]
        ce_o = refs[NPASS + 7:]
        u = jnp.concatenate([r[...] for r in u5], axis=1) * se_r[...]
        m = st_r[0:1] * (1.0 / E)
        var = st_r[1:2] * (1.0 / E) - m * m
        inv = 1.0 / jnp.sqrt(var + EPS)
        r = jnp.maximum((u - m) * inv * g_r[...] + b_r[...], 0.0)
        ce = jnp.dot(r, W_r[...],
                     preferred_element_type=jnp.float32) + xe_r[...] * g2_r[...]
        for p in range(NPASS):
            ce_o[p][...] = ce[:, PF * p:PF * p + PF]

    nb = E // BE
    return pl.pallas_call(
        body,
        grid=(nb,),
        in_specs=[pl.BlockSpec((BE, PF), lambda i: (i, 0))
                  for _ in range(NPASS)]
        + [pl.BlockSpec((BE, 1), lambda i: (i, 0)),
           pl.BlockSpec((BE, 1), lambda i: (i, 0)),
           pl.BlockSpec((2, HP), lambda i: (0, 0)),
           pl.BlockSpec((1, HP), lambda i: (0, 0)),
           pl.BlockSpec((1, HP), lambda i: (0, 0)),
           pl.BlockSpec((HP, HP), lambda i: (0, 0)),
           pl.BlockSpec((1, HP), lambda i: (0, 0))],
        out_specs=[pl.BlockSpec((BE, PF), lambda i: (i, 0))
                   for _ in range(NPASS)],
        out_shape=[jax.ShapeDtypeStruct((EP, PF), jnp.float32)
                   for _ in range(NPASS)],
    )(*us, xe2, se2, estats, g, b, W22, g2)


def _tc_final_mean(tmp2, h1, nstats, g, b):
    def body(tmp_r, h1_r, st_r, g_r, b_r, o_ref):
        i = pl.program_id(0)
        m = st_r[0:1] * (1.0 / N)
        var = st_r[1:2] * (1.0 / N) - m * m
        inv = 1.0 / jnp.sqrt(var + EPS)
        h2 = h1_r[...] + jnp.maximum(
            (tmp_r[...] - m) * inv * g_r[...] + b_r[...], 0.0)
        @pl.when(i == 0)
        def _():
            o_ref[...] = jnp.zeros((1, HP), jnp.float32)
        o_ref[...] = o_ref[...] + jnp.sum(h2, 0, keepdims=True)
        @pl.when(i == (N // BM) - 1)
        def _():
            o_ref[...] = o_ref[...] * (1.0 / N)

    nb = N // BM
    return pl.pallas_call(
        body,
        grid=(nb,),
        in_specs=[
            pl.BlockSpec((BM, HP), lambda i: (i, 0)),
            pl.BlockSpec((BM, HP), lambda i: (i, 0)),
            pl.BlockSpec((2, HP), lambda i: (0, 0)),
            pl.BlockSpec((1, HP), lambda i: (0, 0)),
            pl.BlockSpec((1, HP), lambda i: (0, 0)),
        ],
        out_specs=pl.BlockSpec((1, HP), lambda i: (0, 0)),
        out_shape=jax.ShapeDtypeStruct((1, HP), jnp.float32),
    )(tmp2, h1, nstats, g, b)


def _pairs_i32(acc_bf):
    return jax.lax.bitcast_convert_type(
        acc_bf.reshape(2, NP, PF, 2), jnp.int32)


def _padw(a, shape):
    out = jnp.zeros(shape, jnp.float32)
    if a.ndim == 1:
        return out.at[:a.shape[0]].set(a)
    return out.at[:a.shape[0], :a.shape[1]].set(a)


def kernel(nodes_feat, edges_feat, nodes_num_norm_sqrt, edges_num_norm_sqrt,
           edge_index, emb_h_w, emb_h_b, emb_e_w, emb_e_b,
           l1_W, l1_b, l1_bng_h, l1_bnb_h, l1_bng_e, l1_bnb_e,
           lo_W, lo_b, lo_bng_h, lo_bnb_h, lo_bng_e, lo_bnb_e):
    xe2 = edges_feat
    se2 = edges_num_norm_sqrt
    sn = nodes_num_norm_sqrt
    we = emb_e_w[0]

    xep = jnp.zeros((EP,), jnp.float32).at[:E].set(edges_feat.reshape(E))
    eip = jnp.full((2, EP), N, jnp.int32).at[:, :E].set(edge_index)
    eip = eip.at[0, E:].set(0)
    ei3 = eip.reshape(2, EP // 128, 128)

    Wh = _padw(emb_h_w, (IN_DIM, HP))
    bh = _padw(emb_h_b, (HP,)).reshape(1, HP)
    def catW(Wl, bl):
        Wc = jnp.zeros((HP, 4 * HP), jnp.float32)
        bc = jnp.zeros((4 * HP,), jnp.float32)
        for k, idx in enumerate((0, 1, 3, 4)):
            Wc = Wc.at[:H, k * HP:k * HP + H].set(Wl[idx])
            bc = bc.at[k * HP:k * HP + H].set(bl[idx])
        return Wc, bc.reshape(1, 4 * HP)
    Wc1, bc1 = catW(l1_W, l1_b)
    Wc2, bc2 = catW(lo_W, lo_b)
    c1 = _padw(we @ l1_W[2], (HP,))
    d1 = _padw(emb_e_b @ l1_W[2] + l1_b[2], (HP,)).reshape(1, HP)
    W22 = _padw(lo_W[2], (HP, HP))
    g2 = _padw(we @ lo_W[2], (HP,)).reshape(1, HP)
    d2 = _padw(emb_e_b @ lo_W[2] + lo_b[2], (HP,)).reshape(1, HP)
    g1h = _padw(l1_bng_h, (HP,)).reshape(1, HP)
    b1h = _padw(l1_bnb_h, (HP,)).reshape(1, HP)
    g1e = _padw(l1_bng_e, (HP,)).reshape(1, HP)
    b1e = _padw(l1_bnb_e, (HP,)).reshape(1, HP)
    g2h = _padw(lo_bng_h, (HP,)).reshape(1, HP)
    b2h = _padw(lo_bnb_h, (HP,)).reshape(1, HP)

    prep = _tc_prep(nodes_feat, Wh, bh, Wc1, bc1, d1)
    h0, Ah1 = prep[0], prep[1]
    src1 = prep[2:2 + NPASS]
    dst1 = prep[2 + NPASS:2 + 2 * NPASS]

    accs1, us = [], []
    tok = h0
    for p in range(NPASS):
        acc, u = _sc_pass_l1(ei3, xep, src1[p], dst1[p],
                             lax.dynamic_slice(c1, (PF * p,), (PF,)), tok)
        accs1.append(acc)
        us.append(u)
        tok = acc

    accs1 = [_pairs_i32(a) for a in accs1]
    tmp1, nst1 = _tc_node_gate(accs1, Ah1, sn)
    est = _tc_edge_stats(us, se2)
    upd = _tc_node_update(tmp1, h0, nst1, g1h, b1h, Wc2, bc2, d2)
    h1, Ah2 = upd[0], upd[1]
    src2 = upd[2:2 + NPASS]
    dst2 = upd[2 + NPASS:2 + 2 * NPASS]
    ces = _tc_edge_ce(us, xe2, se2, est, g1e, b1e, W22, g2)

    accs2 = []
    tok = tmp1
    for p in range(NPASS):
        acc = _sc_pass_l2(ei3, ces[p], src2[p], dst2[p], tok)
        accs2.append(acc)
        tok = acc

    accs2 = [_pairs_i32(a) for a in accs2]
    tmp2, nst2 = _tc_node_gate(accs2, Ah2, sn)
    hg = _tc_final_mean(tmp2, h1, nst2, g2h, b2h)
    return hg[:, :H]
